# baseline (speedup 1.0000x reference)
import jax
import jax.numpy as jnp
from jax.experimental import pallas as pl

_ROWS_PER_BLOCK = 2048


def _body(cube_ref, const_ref, out_ref):
    x = cube_ref[...]
    keep = jnp.max(x, axis=-1, keepdims=True) == 1.0
    out_ref[...] = jnp.where(keep, x, const_ref[...])


def kernel(cayley_cube, constant_distribution):
    b, n, _, c = cayley_cube.shape
    rows = b * n * n
    flat = cayley_cube.reshape(rows, c)
    const = constant_distribution.reshape(1, c)
    grid = rows // _ROWS_PER_BLOCK
    out = pl.pallas_call(
        _body,
        grid=(grid,),
        in_specs=[
            pl.BlockSpec((_ROWS_PER_BLOCK, c), lambda i: (i, 0)),
            pl.BlockSpec((1, c), lambda i: (0, 0)),
        ],
        out_specs=pl.BlockSpec((_ROWS_PER_BLOCK, c), lambda i: (i, 0)),
        out_shape=jax.ShapeDtypeStruct((rows, c), cayley_cube.dtype),
    )(flat, const)
    return out.reshape(b, n, n, c)


# speedup vs baseline: 1.6029x; 1.6029x over previous
import jax
import jax.numpy as jnp
from jax.experimental import pallas as pl

_ROWS_PER_BLOCK = 8192


def _body(cube_ref, const_ref, out_ref):
    x = cube_ref[...]
    keep = jnp.max(x, axis=-1, keepdims=True) == 1.0
    out_ref[...] = jnp.where(keep, x, const_ref[...])


def kernel(cayley_cube, constant_distribution):
    b, n, _, c = cayley_cube.shape
    rows = b * n * n
    flat = cayley_cube.reshape(rows, c)
    const = constant_distribution.reshape(1, c)
    grid = rows // _ROWS_PER_BLOCK
    out = pl.pallas_call(
        _body,
        grid=(grid,),
        in_specs=[
            pl.BlockSpec((_ROWS_PER_BLOCK, c), lambda i: (i, 0)),
            pl.BlockSpec((1, c), lambda i: (0, 0)),
        ],
        out_specs=pl.BlockSpec((_ROWS_PER_BLOCK, c), lambda i: (i, 0)),
        out_shape=jax.ShapeDtypeStruct((rows, c), cayley_cube.dtype),
    )(flat, const)
    return out.reshape(b, n, n, c)


# speedup vs baseline: 1.6522x; 1.0307x over previous
import jax
import jax.numpy as jnp
from jax.experimental import pallas as pl

_ROWS_PER_BLOCK = 16384


def _body(cube_ref, const_ref, out_ref):
    x = cube_ref[...]
    keep = jnp.max(x, axis=-1, keepdims=True) == 1.0
    out_ref[...] = jnp.where(keep, x, const_ref[...])


def kernel(cayley_cube, constant_distribution):
    b, n, _, c = cayley_cube.shape
    rows = b * n * n
    flat = cayley_cube.reshape(rows, c)
    const = constant_distribution.reshape(1, c)
    grid = rows // _ROWS_PER_BLOCK
    out = pl.pallas_call(
        _body,
        grid=(grid,),
        in_specs=[
            pl.BlockSpec((_ROWS_PER_BLOCK, c), lambda i: (i, 0)),
            pl.BlockSpec((1, c), lambda i: (0, 0)),
        ],
        out_specs=pl.BlockSpec((_ROWS_PER_BLOCK, c), lambda i: (i, 0)),
        out_shape=jax.ShapeDtypeStruct((rows, c), cayley_cube.dtype),
    )(flat, const)
    return out.reshape(b, n, n, c)
